# baseline (speedup 1.0000x reference)
import functools

import jax
import jax.numpy as jnp
from jax import lax
from jax.experimental import pallas as pl
from jax.experimental.pallas import tpu as pltpu
from jax.experimental.pallas import tpu_sc as plsc

BATCH = 16384
SEQ = 50
DIM = 64
PAD = 128
NUM_IDX = BATCH * SEQ

_info = plsc.get_sparse_core_info()
_NC, _NS = _info.num_cores, _info.num_subcores
NW = _NC * _NS

B_PER_W = NUM_IDX // NW
CHUNK = 128
K = 5
BLK = K * CHUNK
N_BLOCKS = B_PER_W // BLK
N_PAIRS = N_BLOCKS // 2

ROWS_PER_W = BATCH // NW
RB = 8
FB = RB * SEQ
F_BLOCKS = ROWS_PER_W // RB
F_PAIRS = F_BLOCKS // 2

_mesh = plsc.VectorSubcoreMesh(core_axis_name="c", subcore_axis_name="s")


@functools.partial(
    pl.kernel,
    mesh=_mesh,
    out_type=jax.ShapeDtypeStruct((NUM_IDX, DIM), jnp.float32),
    scratch_types=[
        pltpu.VMEM((B_PER_W,), jnp.int32),
        pltpu.VMEM((BLK, DIM), jnp.float32),
        pltpu.VMEM((BLK, DIM), jnp.float32),
        pltpu.SemaphoreType.DMA,
        pltpu.SemaphoreType.DMA,
        pltpu.SemaphoreType.DMA,
    ],
    compiler_params=pltpu.CompilerParams(use_tc_tiling_on_sc=False),
)
def _sc_gather(idx_hbm, table_hbm, out_hbm, idx_all, rows0, rows1,
               gsem, ssem0, ssem1):
    wid = lax.axis_index("s") * _NC + lax.axis_index("c")
    base = wid * B_PER_W

    pltpu.sync_copy(idx_hbm.at[pl.ds(base, B_PER_W)], idx_all)

    def fire_gathers(g, rows):
        for j in range(K):
            pltpu.async_copy(
                table_hbm.at[idx_all.at[pl.ds(g * BLK + j * CHUNK, CHUNK)]],
                rows.at[pl.ds(j * CHUNK, CHUNK)],
                gsem,
            )

    def wait_gathers(rows):
        pltpu.make_async_copy(table_hbm.at[pl.ds(0, BLK)], rows, gsem).wait()

    def fire_store(g, rows, sem):
        pltpu.async_copy(rows, out_hbm.at[pl.ds(base + g * BLK, BLK)], sem)

    def wait_store(rows, sem):
        pltpu.make_async_copy(rows, out_hbm.at[pl.ds(base, BLK)], sem).wait()

    fire_gathers(0, rows0)

    def body(p, carry):
        g0 = 2 * p
        wait_gathers(rows0)
        fire_store(g0, rows0, ssem0)

        @pl.when(p > 0)
        def _():
            wait_store(rows1, ssem1)

        fire_gathers(g0 + 1, rows1)
        wait_gathers(rows1)
        fire_store(g0 + 1, rows1, ssem1)

        @pl.when(p < N_PAIRS - 1)
        def _():
            wait_store(rows0, ssem0)
            fire_gathers(g0 + 2, rows0)

        return carry

    lax.fori_loop(0, N_PAIRS, body, 0)
    wait_store(rows0, ssem0)
    wait_store(rows1, ssem1)


@functools.partial(
    pl.kernel,
    mesh=_mesh,
    out_type=jax.ShapeDtypeStruct((BATCH, SEQ, DIM), jnp.float32),
    scratch_types=[
        pltpu.VMEM((FB, DIM), jnp.float32),
        pltpu.VMEM((FB, DIM), jnp.float32),
        pltpu.SemaphoreType.DMA,
        pltpu.SemaphoreType.DMA,
        pltpu.SemaphoreType.DMA,
        pltpu.SemaphoreType.DMA,
    ],
)
def _sc_tilize(mid_hbm, out_hbm, buf0, buf1, lsem0, lsem1, ssem0, ssem1):
    wid = lax.axis_index("s") * _NC + lax.axis_index("c")
    base = wid * ROWS_PER_W

    def fire_load(g, buf, sem):
        pltpu.async_copy(mid_hbm.at[pl.ds((base + g * RB) * SEQ, FB)], buf, sem)

    def wait_load(buf, sem):
        pltpu.make_async_copy(mid_hbm.at[pl.ds(0, FB)], buf, sem).wait()

    def fire_stores(g, buf, sem):
        for r in range(RB):
            pltpu.async_copy(buf.at[pl.ds(r * SEQ, SEQ)],
                             out_hbm.at[base + g * RB + r], sem)

    def wait_stores(buf, sem):
        for r in range(RB):
            pltpu.make_async_copy(buf.at[pl.ds(r * SEQ, SEQ)],
                                  out_hbm.at[base + r], sem).wait()

    fire_load(0, buf0, lsem0)

    def body(p, carry):
        g0 = 2 * p
        wait_load(buf0, lsem0)
        fire_stores(g0, buf0, ssem0)

        @pl.when(p > 0)
        def _():
            wait_stores(buf1, ssem1)

        fire_load(g0 + 1, buf1, lsem1)
        wait_load(buf1, lsem1)
        fire_stores(g0 + 1, buf1, ssem1)

        @pl.when(p < F_PAIRS - 1)
        def _():
            wait_stores(buf0, ssem0)
            fire_load(g0 + 2, buf0, lsem0)

        return carry

    lax.fori_loop(0, F_PAIRS, body, 0)
    wait_stores(buf0, ssem0)
    wait_stores(buf1, ssem1)


def kernel(indices, weight):
    flat = indices.reshape(-1).astype(jnp.int32)
    mid = _sc_gather(flat, weight)
    return _sc_tilize(mid)


# speedup vs baseline: 1.2235x; 1.2235x over previous
import functools

import jax
import jax.numpy as jnp
from jax import lax
from jax.experimental import pallas as pl
from jax.experimental.pallas import tpu as pltpu
from jax.experimental.pallas import tpu_sc as plsc

BATCH = 16384
SEQ = 50
DIM = 64
PAD = 128
NUM_IDX = BATCH * SEQ

_info = plsc.get_sparse_core_info()
_NC, _NS = _info.num_cores, _info.num_subcores
NW = _NC * _NS

B_PER_W = NUM_IDX // NW
CHUNK = 128
K = 5
BLK = K * CHUNK
N_BLOCKS = B_PER_W // BLK
N_PAIRS = N_BLOCKS // 2

ROWS_PER_W = BATCH // NW
RB = 4
FB = RB * SEQ
F_BLOCKS = ROWS_PER_W // RB
F_PAIRS = F_BLOCKS // 2
RR_UNROLL = 8

_mesh = plsc.VectorSubcoreMesh(core_axis_name="c", subcore_axis_name="s")


@functools.partial(
    pl.kernel,
    mesh=_mesh,
    out_type=jax.ShapeDtypeStruct((NUM_IDX, PAD), jnp.float32),
    scratch_types=[
        pltpu.VMEM((B_PER_W,), jnp.int32),
        pltpu.VMEM((BLK, DIM), jnp.float32),
        pltpu.VMEM((BLK, DIM), jnp.float32),
        pltpu.SemaphoreType.DMA,
        pltpu.SemaphoreType.DMA,
        pltpu.SemaphoreType.DMA,
    ],
    compiler_params=pltpu.CompilerParams(use_tc_tiling_on_sc=False),
)
def _sc_gather(idx_hbm, table_hbm, out_hbm, idx_all, rows0, rows1,
               gsem, ssem0, ssem1):
    wid = lax.axis_index("s") * _NC + lax.axis_index("c")
    base = wid * B_PER_W

    pltpu.sync_copy(idx_hbm.at[pl.ds(base, B_PER_W)], idx_all)

    def fire_gathers(g, rows):
        for j in range(K):
            pltpu.async_copy(
                table_hbm.at[idx_all.at[pl.ds(g * BLK + j * CHUNK, CHUNK)]],
                rows.at[pl.ds(j * CHUNK, CHUNK)],
                gsem,
            )

    def wait_gathers(rows):
        pltpu.make_async_copy(table_hbm.at[pl.ds(0, BLK)], rows, gsem).wait()

    def fire_store(g, rows, sem):
        pltpu.async_copy(
            rows, out_hbm.at[pl.ds(base + g * BLK, BLK), pl.ds(0, DIM)], sem)

    def wait_store(rows, sem):
        pltpu.make_async_copy(
            rows, out_hbm.at[pl.ds(base, BLK), pl.ds(0, DIM)], sem).wait()

    fire_gathers(0, rows0)

    def body(p, carry):
        g0 = 2 * p
        wait_gathers(rows0)
        fire_store(g0, rows0, ssem0)

        @pl.when(p > 0)
        def _():
            wait_store(rows1, ssem1)

        fire_gathers(g0 + 1, rows1)
        wait_gathers(rows1)
        fire_store(g0 + 1, rows1, ssem1)

        @pl.when(p < N_PAIRS - 1)
        def _():
            wait_store(rows0, ssem0)
            fire_gathers(g0 + 2, rows0)

        return carry

    lax.fori_loop(0, N_PAIRS, body, 0)
    wait_store(rows0, ssem0)
    wait_store(rows1, ssem1)


@functools.partial(
    pl.kernel,
    mesh=_mesh,
    out_type=jax.ShapeDtypeStruct((BATCH, SEQ, DIM), jnp.float32),
    scratch_types=[
        pltpu.VMEM((FB, PAD), jnp.float32),
        pltpu.VMEM((FB, PAD), jnp.float32),
        pltpu.VMEM((FB, DIM), jnp.float32),
        pltpu.VMEM((FB, DIM), jnp.float32),
        pltpu.SemaphoreType.DMA,
        pltpu.SemaphoreType.DMA,
        pltpu.SemaphoreType.DMA,
        pltpu.SemaphoreType.DMA,
    ],
)
def _sc_tilize(mid_hbm, out_hbm, wide0, wide1, nar0, nar1,
               lsem0, lsem1, ssem0, ssem1):
    wid = lax.axis_index("s") * _NC + lax.axis_index("c")
    base = wid * ROWS_PER_W

    def fire_load(g, wide, sem):
        pltpu.async_copy(
            mid_hbm.at[pl.ds((base + g * RB) * SEQ, FB)], wide, sem)

    def wait_load(wide, sem):
        pltpu.make_async_copy(mid_hbm.at[pl.ds(0, FB)], wide, sem).wait()

    def repack(wide, nar):
        def rows(i, carry):
            for rr in range(RR_UNROLL):
                for c in range(DIM // 16):
                    nar[i * RR_UNROLL + rr, pl.ds(c * 16, 16)] = (
                        wide[i * RR_UNROLL + rr, pl.ds(c * 16, 16)])
            return carry

        lax.fori_loop(0, FB // RR_UNROLL, rows, 0)

    def fire_stores(g, nar, sem):
        for r in range(RB):
            pltpu.async_copy(nar.at[pl.ds(r * SEQ, SEQ)],
                             out_hbm.at[base + g * RB + r], sem)

    def wait_stores(nar, sem):
        for r in range(RB):
            pltpu.make_async_copy(nar.at[pl.ds(r * SEQ, SEQ)],
                                  out_hbm.at[base + r], sem).wait()

    fire_load(0, wide0, lsem0)

    def body(p, carry):
        g0 = 2 * p
        wait_load(wide0, lsem0)

        @pl.when(p > 0)
        def _():
            wait_stores(nar0, ssem0)

        fire_load(g0 + 1, wide1, lsem1)
        repack(wide0, nar0)
        fire_stores(g0, nar0, ssem0)

        wait_load(wide1, lsem1)

        @pl.when(p > 0)
        def _():
            wait_stores(nar1, ssem1)

        @pl.when(p < F_PAIRS - 1)
        def _():
            fire_load(g0 + 2, wide0, lsem0)

        repack(wide1, nar1)
        fire_stores(g0 + 1, nar1, ssem1)
        return carry

    lax.fori_loop(0, F_PAIRS, body, 0)
    wait_stores(nar0, ssem0)
    wait_stores(nar1, ssem1)


def kernel(indices, weight):
    flat = indices.reshape(-1).astype(jnp.int32)
    mid = _sc_gather(flat, weight)
    return _sc_tilize(mid)


# speedup vs baseline: 1.3246x; 1.0826x over previous
import functools

import jax
import jax.numpy as jnp
from jax import lax
from jax.experimental import pallas as pl
from jax.experimental.pallas import tpu as pltpu
from jax.experimental.pallas import tpu_sc as plsc

NUM_IDX = 16384 * 50
DIM = 64

_info = plsc.get_sparse_core_info()
_NC, _NS = _info.num_cores, _info.num_subcores
NW = _NC * _NS
B_PER_W = NUM_IDX // NW
CHUNK = 128
K = 5
BLK = K * CHUNK
N_BLOCKS = B_PER_W // BLK
N_PAIRS = N_BLOCKS // 2

_mesh = plsc.VectorSubcoreMesh(core_axis_name="c", subcore_axis_name="s")


@functools.partial(
    pl.kernel,
    mesh=_mesh,
    out_type=jax.ShapeDtypeStruct((NUM_IDX, DIM), jnp.float32),
    scratch_types=[
        pltpu.VMEM((B_PER_W,), jnp.int32),
        pltpu.VMEM((BLK, DIM), jnp.float32),
        pltpu.VMEM((BLK, DIM), jnp.float32),
        pltpu.SemaphoreType.DMA,
        pltpu.SemaphoreType.DMA,
        pltpu.SemaphoreType.DMA,
    ],
    compiler_params=pltpu.CompilerParams(use_tc_tiling_on_sc=False),
)
def _sc_gather(idx_hbm, table_hbm, out_hbm, idx_all, rows0, rows1,
               gsem, ssem0, ssem1):
    wid = lax.axis_index("s") * _NC + lax.axis_index("c")
    base = wid * B_PER_W

    pltpu.sync_copy(idx_hbm.at[pl.ds(base, B_PER_W)], idx_all)

    def fire_gathers(g, rows):
        for j in range(K):
            pltpu.async_copy(
                table_hbm.at[idx_all.at[pl.ds(g * BLK + j * CHUNK, CHUNK)]],
                rows.at[pl.ds(j * CHUNK, CHUNK)],
                gsem,
            )

    def wait_gathers(rows):
        pltpu.make_async_copy(out_hbm.at[pl.ds(base, BLK)], rows, gsem).wait()

    def fire_store(g, rows, sem):
        pltpu.async_copy(rows, out_hbm.at[pl.ds(base + g * BLK, BLK)], sem)

    def wait_store(rows, sem):
        pltpu.make_async_copy(rows, out_hbm.at[pl.ds(base, BLK)], sem).wait()

    fire_gathers(0, rows0)

    def body(p, carry):
        g0 = 2 * p
        wait_gathers(rows0)
        fire_store(g0, rows0, ssem0)

        @pl.when(p > 0)
        def _():
            wait_store(rows1, ssem1)

        fire_gathers(g0 + 1, rows1)
        wait_gathers(rows1)
        fire_store(g0 + 1, rows1, ssem1)

        @pl.when(p < N_PAIRS - 1)
        def _():
            wait_store(rows0, ssem0)
            fire_gathers(g0 + 2, rows0)

        return carry

    lax.fori_loop(0, N_PAIRS, body, 0)
    wait_store(rows0, ssem0)
    wait_store(rows1, ssem1)


def kernel(indices, weight):
    flat = indices.reshape(-1).astype(jnp.int32)
    out = _sc_gather(flat, weight)
    return out.reshape(indices.shape + (weight.shape[1],))
